# baseline (speedup 1.0000x reference)
import functools

import jax
import jax.numpy as jnp
from jax import lax
from jax.experimental import pallas as pl
from jax.experimental.pallas import tpu as pltpu
from jax.experimental.pallas import tpu_sc as plsc

N = 10000
D = 256
E = 160000
NT = 5
K = 64
EPAD = 172032
EROWS = EPAD // K
BPT = EPAD // (16 * K)
HALF = N // 2
ACC_ROWS = 5024
ZCHUNK = ACC_ROWS // 16
TRASH = 5008
WCHUNK = 313
PADDST = 2 * N
EPS = 1e-5
MB = 1000


def _idx_body(src_ref, dst_ref, typ_ref, p_ref):
    p_ref[...] = dst_ref[...] * 65536 + (typ_ref[...] * N + src_ref[...])


def _idx_prep(srcp, dstp, typp):
    grid = EROWS // 8
    return pl.pallas_call(
        _idx_body,
        grid=(grid,),
        in_specs=[pl.BlockSpec((8, K), lambda i: (i, 0))] * 3,
        out_specs=pl.BlockSpec((8, K), lambda i: (i, 0)),
        out_shape=jax.ShapeDtypeStruct((EROWS, K), jnp.int32),
    )(srcp, dstp, typp)


def _ybuild_body(x_ref, w_ref, y_ref):
    t = pl.program_id(0)
    w = w_ref[pl.ds(t, 1), :]
    y_ref[...] = jnp.maximum(x_ref[...] + w, 0.0)


def _ybuild(x, w_type):
    yb = 1000
    nb = N // yb
    return pl.pallas_call(
        _ybuild_body,
        grid=(NT, nb),
        in_specs=[
            pl.BlockSpec((yb, D), lambda t, i: (i, 0)),
            pl.BlockSpec((8, D), lambda t, i: (0, 0)),
        ],
        out_specs=pl.BlockSpec((yb, D), lambda t, i: (t * nb + i, 0)),
        out_shape=jax.ShapeDtypeStruct((NT * N, D), jnp.float32),
    )(x, w_type)


_SC_MESH = plsc.VectorSubcoreMesh(core_axis_name="c", subcore_axis_name="s")


@functools.partial(
    pl.kernel,
    mesh=_SC_MESH,
    out_type=jax.ShapeDtypeStruct((N, D), jnp.float32),
    scratch_types=[
        pltpu.VMEM((BPT * K + 2 * K,), jnp.int32),
        pltpu.VMEM((16,), jnp.int32),
        pltpu.VMEM((2, K), jnp.int32),
        pltpu.VMEM((2, K), jnp.int32),
        pltpu.VMEM((2, K, D), jnp.float32),
        pltpu.VMEM_SHARED((ACC_ROWS, D), jnp.float32),
        pltpu.SemaphoreType.DMA,
        pltpu.SemaphoreType.DMA,
    ],
    compiler_params=pltpu.CompilerParams(use_tc_tiling_on_sc=False,
                                        needs_layout_passes=False),
)
def _sc_agg(y_hbm, pix_hbm, out_hbm,
            pix_v, cnt_v, gixb, dixb, rows_v, acc_sh, sem0, sem1):
    cid = lax.axis_index("c")
    sid = lax.axis_index("s")
    base = cid * HALF

    def zbody(j, c):
        rows_v[0, j // 16, pl.ds((j % 16) * 16, 16)] = jnp.zeros((16,),
                                                                 jnp.float32)
        return c

    lax.fori_loop(0, K * 16, zbody, 0)
    zb = sid * ZCHUNK
    for o in range(0, ZCHUNK - K + 1, K):
        pltpu.sync_copy(rows_v.at[0], acc_sh.at[pl.ds(zb + o, K)])
    rem = ZCHUNK % K
    if rem:
        pltpu.sync_copy(rows_v.at[0, pl.ds(0, rem)],
                        acc_sh.at[pl.ds(zb + ZCHUNK - rem, rem)])
    pltpu.sync_copy(pix_hbm.at[pl.ds(sid * BPT * K, BPT * K)],
                    pix_v.at[pl.ds(0, BPT * K)])

    def cbody(j, cnt):
        v = pix_v[pl.ds(j * 16, 16)]
        d = lax.shift_right_logical(v, 16)
        m = (d >= base) & (d < base + HALF)
        mi = m.astype(jnp.int32)
        return cnt + 1

    cntv = lax.fori_loop(0, BPT * K // 16, cbody, 0)
    probe = jnp.max(pix_v[pl.ds(0, 16)]) * 0
    npair = BPT // 2 + probe
    plsc.subcore_barrier()

    def unpack(b, slot):
        for w in range(K // 16):
            v = pix_v[pl.ds(b * K + w * 16, 16)]
            g = v & 0xFFFF
            d = lax.shift_right_logical(v, 16)
            inr = (d >= base) & (d < base + HALF)
            gixb[slot, pl.ds(w * 16, 16)] = g
            dixb[slot, pl.ds(w * 16, 16)] = jnp.where(inr, d - base, TRASH)

    def body(g, c):
        b0 = 2 * g
        unpack(b0, 0)
        cp0 = pltpu.async_copy(y_hbm.at[gixb.at[0]], rows_v.at[0], sem0)
        unpack(b0 + 1, 1)
        cp1 = pltpu.async_copy(y_hbm.at[gixb.at[1]], rows_v.at[1], sem1)
        cp0.wait()
        pltpu.sync_copy(rows_v.at[0], acc_sh.at[dixb.at[0]], add=True)
        cp1.wait()
        pltpu.sync_copy(rows_v.at[1], acc_sh.at[dixb.at[1]], add=True)
        return c

    lax.fori_loop(0, npair, body, 0)
    plsc.subcore_barrier()
    start = jnp.minimum(sid * WCHUNK, HALF - WCHUNK)
    pltpu.sync_copy(acc_sh.at[pl.ds(start, WCHUNK)],
                    out_hbm.at[pl.ds(cid * HALF + start, WCHUNK)])


def _mlp_body(a_ref, w1_ref, b1_ref, w2_ref, b2_ref, h_ref, st_ref):
    i = pl.program_id(0)
    a = a_ref[...]
    h1 = lax.dot_general(a, w1_ref[...], (((1,), (1,)), ((), ())),
                         preferred_element_type=jnp.float32)
    h1 = jnp.maximum(h1 + b1_ref[...], 0.0)
    h = lax.dot_general(h1, w2_ref[...], (((1,), (1,)), ((), ())),
                        preferred_element_type=jnp.float32)
    h = h + b2_ref[...]
    h_ref[...] = h

    @pl.when(i == 0)
    def _():
        st_ref[...] = jnp.zeros_like(st_ref)

    st_ref[0:1, :] += jnp.sum(h, axis=0, keepdims=True)
    st_ref[1:2, :] += jnp.sum(h * h, axis=0, keepdims=True)


def _mlp(aggr, w1, b1, w2, b2):
    return pl.pallas_call(
        _mlp_body,
        grid=(N // MB,),
        in_specs=[
            pl.BlockSpec((MB, D), lambda i: (i, 0)),
            pl.BlockSpec((2 * D, D), lambda i: (0, 0)),
            pl.BlockSpec((1, 2 * D), lambda i: (0, 0)),
            pl.BlockSpec((D, 2 * D), lambda i: (0, 0)),
            pl.BlockSpec((1, D), lambda i: (0, 0)),
        ],
        out_specs=[
            pl.BlockSpec((MB, D), lambda i: (i, 0)),
            pl.BlockSpec((8, D), lambda i: (0, 0)),
        ],
        out_shape=[
            jax.ShapeDtypeStruct((N, D), jnp.float32),
            jax.ShapeDtypeStruct((8, D), jnp.float32),
        ],
    )(aggr, w1, b1, w2, b2)


def _norm_body(h_ref, st_ref, gam_ref, bet_ref, o_ref):
    mean = st_ref[0:1, :] * (1.0 / N)
    var = st_ref[1:2, :] * (1.0 / N) - mean * mean
    inv = lax.rsqrt(var + EPS)
    o_ref[...] = jnp.maximum(
        (h_ref[...] - mean) * inv * gam_ref[...] + bet_ref[...], 0.0)


def _norm(h, st, gamma, beta):
    return pl.pallas_call(
        _norm_body,
        grid=(N // MB,),
        in_specs=[
            pl.BlockSpec((MB, D), lambda i: (i, 0)),
            pl.BlockSpec((8, D), lambda i: (0, 0)),
            pl.BlockSpec((1, D), lambda i: (0, 0)),
            pl.BlockSpec((1, D), lambda i: (0, 0)),
        ],
        out_specs=pl.BlockSpec((MB, D), lambda i: (i, 0)),
        out_shape=jax.ShapeDtypeStruct((N, D), jnp.float32),
    )(h, st, gamma, beta)


def _prep(x, edge_index, edge_attr, w_type):
    ei = edge_index.astype(jnp.int32)
    t = edge_attr[:, 0].astype(jnp.int32)
    loops = jnp.arange(N, dtype=jnp.int32)
    npad = EPAD - E - N
    padz = jnp.zeros((npad,), jnp.int32)
    srcp = jnp.concatenate([ei[0], loops, padz]).reshape(EROWS, K)
    dstp = jnp.concatenate(
        [ei[1], loops, jnp.full((npad,), PADDST, jnp.int32)]).reshape(EROWS, K)
    typp = jnp.concatenate(
        [t, jnp.full((N,), 4, jnp.int32), padz]).reshape(EROWS, K)
    packed = _idx_prep(srcp, dstp, typp).reshape(EPAD)
    y = _ybuild(x, w_type)
    return packed, y


def _post(aggr, w1, b1, w2, b2, gamma, beta):
    h, st = _mlp(aggr, w1, b1.reshape(1, -1), w2, b2.reshape(1, -1))
    return _norm(h, st, gamma.reshape(1, -1), beta.reshape(1, -1))


def kernel(xA, edge_indexA, edge_attrA, xB, edge_indexB, edge_attrB,
           W_type, W1, b1, W2, b2, gamma, beta):
    packedA, yA = _prep(xA, edge_indexA, edge_attrA, W_type)
    packedB, yB = _prep(xB, edge_indexB, edge_attrB, W_type)
    aggrA = _sc_agg(yA, packedA)
    aggrB = _sc_agg(yB, packedB)
    outA = _post(aggrA, W1, b1, W2, b2, gamma, beta)
    outB = _post(aggrB, W1, b1, W2, b2, gamma, beta)
    return (outA, outB)


# speedup vs baseline: 1.0059x; 1.0059x over previous
import functools

import jax
import jax.numpy as jnp
from jax import lax
from jax.experimental import pallas as pl
from jax.experimental.pallas import tpu as pltpu
from jax.experimental.pallas import tpu_sc as plsc

N = 10000
D = 256
E = 160000
NT = 5
K = 64
EPAD = 172032
EROWS = EPAD // K
BPT = EPAD // (16 * K)
HALF = N // 2
ACC_ROWS = 5024
ZCHUNK = ACC_ROWS // 16
TRASH = 5008
WCHUNK = 313
PADDST = 2 * N
EPS = 1e-5
MB = 1000


def _idx_body(src_ref, dst_ref, typ_ref, p_ref):
    p_ref[...] = dst_ref[...] * 65536 + (typ_ref[...] * N + src_ref[...])


def _idx_prep(srcp, dstp, typp):
    grid = EROWS // 8
    return pl.pallas_call(
        _idx_body,
        grid=(grid,),
        in_specs=[pl.BlockSpec((8, K), lambda i: (i, 0))] * 3,
        out_specs=pl.BlockSpec((8, K), lambda i: (i, 0)),
        out_shape=jax.ShapeDtypeStruct((EROWS, K), jnp.int32),
    )(srcp, dstp, typp)


def _ybuild_body(x_ref, w_ref, y_ref):
    t = pl.program_id(0)
    w = w_ref[pl.ds(t, 1), :]
    y_ref[...] = jnp.maximum(x_ref[...] + w, 0.0)


def _ybuild(x, w_type):
    yb = 1000
    nb = N // yb
    return pl.pallas_call(
        _ybuild_body,
        grid=(NT, nb),
        in_specs=[
            pl.BlockSpec((yb, D), lambda t, i: (i, 0)),
            pl.BlockSpec((8, D), lambda t, i: (0, 0)),
        ],
        out_specs=pl.BlockSpec((yb, D), lambda t, i: (t * nb + i, 0)),
        out_shape=jax.ShapeDtypeStruct((NT * N, D), jnp.float32),
    )(x, w_type)


_SC_MESH = plsc.VectorSubcoreMesh(core_axis_name="c", subcore_axis_name="s")


@functools.partial(
    pl.kernel,
    mesh=_SC_MESH,
    out_type=jax.ShapeDtypeStruct((N, D), jnp.float32),
    scratch_types=[
        pltpu.VMEM((BPT * K + 2 * K,), jnp.int32),
        pltpu.VMEM((16,), jnp.int32),
        pltpu.VMEM((2, K), jnp.int32),
        pltpu.VMEM((2, K), jnp.int32),
        pltpu.VMEM((2, K, D), jnp.float32),
        pltpu.VMEM_SHARED((ACC_ROWS, D), jnp.float32),
        pltpu.SemaphoreType.DMA,
        pltpu.SemaphoreType.DMA,
        pltpu.SemaphoreType.DMA,
        pltpu.SemaphoreType.DMA,
    ],
    compiler_params=pltpu.CompilerParams(use_tc_tiling_on_sc=False,
                                        needs_layout_passes=False),
)
def _sc_agg(y_hbm, pix_hbm, out_hbm,
            pix_v, cnt_v, gixb, dixb, rows_v, acc_sh, sem0, sem1, sem2, sem3):
    cid = lax.axis_index("c")
    sid = lax.axis_index("s")
    base = cid * HALF

    def zbody(j, c):
        rows_v[0, j // 16, pl.ds((j % 16) * 16, 16)] = jnp.zeros((16,),
                                                                 jnp.float32)
        return c

    lax.fori_loop(0, K * 16, zbody, 0)
    zb = sid * ZCHUNK
    for o in range(0, ZCHUNK - K + 1, K):
        pltpu.sync_copy(rows_v.at[0], acc_sh.at[pl.ds(zb + o, K)])
    rem = ZCHUNK % K
    if rem:
        pltpu.sync_copy(rows_v.at[0, pl.ds(0, rem)],
                        acc_sh.at[pl.ds(zb + ZCHUNK - rem, rem)])
    pltpu.sync_copy(pix_hbm.at[pl.ds(sid * BPT * K, BPT * K)],
                    pix_v.at[pl.ds(0, BPT * K)])

    def cbody(j, cnt):
        v = pix_v[pl.ds(j * 16, 16)]
        d = lax.shift_right_logical(v, 16)
        m = (d >= base) & (d < base + HALF)
        mi = m.astype(jnp.int32)
        return cnt + 1

    cntv = lax.fori_loop(0, BPT * K // 16, cbody, 0)
    probe = jnp.max(pix_v[pl.ds(0, 16)]) * 0
    npair = BPT // 2 + probe
    plsc.subcore_barrier()

    def unpack(b, slot):
        for w in range(K // 16):
            v = pix_v[pl.ds(b * K + w * 16, 16)]
            g = v & 0xFFFF
            d = lax.shift_right_logical(v, 16)
            inr = (d >= base) & (d < base + HALF)
            gixb[slot, pl.ds(w * 16, 16)] = g
            dixb[slot, pl.ds(w * 16, 16)] = jnp.where(inr, d - base, TRASH)

    def body(g, c):
        b0 = 2 * g
        unpack(b0, 0)
        cp0 = pltpu.async_copy(y_hbm.at[gixb.at[0]], rows_v.at[0], sem0)
        unpack(b0 + 1, 1)
        cp1 = pltpu.async_copy(y_hbm.at[gixb.at[1]], rows_v.at[1], sem1)
        cp0.wait()
        sc0 = pltpu.async_copy(rows_v.at[0], acc_sh.at[dixb.at[0]], sem2,
                               add=True)
        cp1.wait()
        sc1 = pltpu.async_copy(rows_v.at[1], acc_sh.at[dixb.at[1]], sem3,
                               add=True)
        sc0.wait()
        sc1.wait()
        return c

    lax.fori_loop(0, npair, body, 0)
    plsc.subcore_barrier()
    start = jnp.minimum(sid * WCHUNK, HALF - WCHUNK)
    pltpu.sync_copy(acc_sh.at[pl.ds(start, WCHUNK)],
                    out_hbm.at[pl.ds(cid * HALF + start, WCHUNK)])


def _mlp_body(a_ref, w1_ref, b1_ref, w2_ref, b2_ref, h_ref, st_ref):
    i = pl.program_id(0)
    a = a_ref[...]
    h1 = lax.dot_general(a, w1_ref[...], (((1,), (1,)), ((), ())),
                         preferred_element_type=jnp.float32)
    h1 = jnp.maximum(h1 + b1_ref[...], 0.0)
    h = lax.dot_general(h1, w2_ref[...], (((1,), (1,)), ((), ())),
                        preferred_element_type=jnp.float32)
    h = h + b2_ref[...]
    h_ref[...] = h

    @pl.when(i == 0)
    def _():
        st_ref[...] = jnp.zeros_like(st_ref)

    st_ref[0:1, :] += jnp.sum(h, axis=0, keepdims=True)
    st_ref[1:2, :] += jnp.sum(h * h, axis=0, keepdims=True)


def _mlp(aggr, w1, b1, w2, b2):
    return pl.pallas_call(
        _mlp_body,
        grid=(N // MB,),
        in_specs=[
            pl.BlockSpec((MB, D), lambda i: (i, 0)),
            pl.BlockSpec((2 * D, D), lambda i: (0, 0)),
            pl.BlockSpec((1, 2 * D), lambda i: (0, 0)),
            pl.BlockSpec((D, 2 * D), lambda i: (0, 0)),
            pl.BlockSpec((1, D), lambda i: (0, 0)),
        ],
        out_specs=[
            pl.BlockSpec((MB, D), lambda i: (i, 0)),
            pl.BlockSpec((8, D), lambda i: (0, 0)),
        ],
        out_shape=[
            jax.ShapeDtypeStruct((N, D), jnp.float32),
            jax.ShapeDtypeStruct((8, D), jnp.float32),
        ],
    )(aggr, w1, b1, w2, b2)


def _norm_body(h_ref, st_ref, gam_ref, bet_ref, o_ref):
    mean = st_ref[0:1, :] * (1.0 / N)
    var = st_ref[1:2, :] * (1.0 / N) - mean * mean
    inv = lax.rsqrt(var + EPS)
    o_ref[...] = jnp.maximum(
        (h_ref[...] - mean) * inv * gam_ref[...] + bet_ref[...], 0.0)


def _norm(h, st, gamma, beta):
    return pl.pallas_call(
        _norm_body,
        grid=(N // MB,),
        in_specs=[
            pl.BlockSpec((MB, D), lambda i: (i, 0)),
            pl.BlockSpec((8, D), lambda i: (0, 0)),
            pl.BlockSpec((1, D), lambda i: (0, 0)),
            pl.BlockSpec((1, D), lambda i: (0, 0)),
        ],
        out_specs=pl.BlockSpec((MB, D), lambda i: (i, 0)),
        out_shape=jax.ShapeDtypeStruct((N, D), jnp.float32),
    )(h, st, gamma, beta)


def _prep(x, edge_index, edge_attr, w_type):
    ei = edge_index.astype(jnp.int32)
    t = edge_attr[:, 0].astype(jnp.int32)
    loops = jnp.arange(N, dtype=jnp.int32)
    npad = EPAD - E - N
    padz = jnp.zeros((npad,), jnp.int32)
    srcp = jnp.concatenate([ei[0], loops, padz]).reshape(EROWS, K)
    dstp = jnp.concatenate(
        [ei[1], loops, jnp.full((npad,), PADDST, jnp.int32)]).reshape(EROWS, K)
    typp = jnp.concatenate(
        [t, jnp.full((N,), 4, jnp.int32), padz]).reshape(EROWS, K)
    packed = _idx_prep(srcp, dstp, typp).reshape(EPAD)
    y = _ybuild(x, w_type)
    return packed, y


def _post(aggr, w1, b1, w2, b2, gamma, beta):
    h, st = _mlp(aggr, w1, b1.reshape(1, -1), w2, b2.reshape(1, -1))
    return _norm(h, st, gamma.reshape(1, -1), beta.reshape(1, -1))


def kernel(xA, edge_indexA, edge_attrA, xB, edge_indexB, edge_attrB,
           W_type, W1, b1, W2, b2, gamma, beta):
    packedA, yA = _prep(xA, edge_indexA, edge_attrA, W_type)
    packedB, yB = _prep(xB, edge_indexB, edge_attrB, W_type)
    aggrA = _sc_agg(yA, packedA)
    aggrB = _sc_agg(yB, packedB)
    outA = _post(aggrA, W1, b1, W2, b2, gamma, beta)
    outB = _post(aggrB, W1, b1, W2, b2, gamma, beta)
    return (outA, outB)


# speedup vs baseline: 1.1248x; 1.1182x over previous
import functools

import jax
import jax.numpy as jnp
from jax import lax
from jax.experimental import pallas as pl
from jax.experimental.pallas import tpu as pltpu
from jax.experimental.pallas import tpu_sc as plsc

N = 10000
D = 256
E = 160000
NT = 5
K = 64
EPAD = 172032
EROWS = EPAD // K
BPT = EPAD // (16 * K)
HALF = N // 2
ACC_ROWS = 5024
ZCHUNK = ACC_ROWS // 16
TRASH = 5008
WCHUNK = 313
PADDST = 2 * N
EPS = 1e-5
MB = 1000


def _idx_body(src_ref, dst_ref, typ_ref, p_ref):
    p_ref[...] = dst_ref[...] * 65536 + (typ_ref[...] * N + src_ref[...])


def _idx_prep(srcp, dstp, typp):
    grid = EROWS // 8
    return pl.pallas_call(
        _idx_body,
        grid=(grid,),
        in_specs=[pl.BlockSpec((8, K), lambda i: (i, 0))] * 3,
        out_specs=pl.BlockSpec((8, K), lambda i: (i, 0)),
        out_shape=jax.ShapeDtypeStruct((EROWS, K), jnp.int32),
    )(srcp, dstp, typp)


def _ybuild_body(x_ref, w_ref, y_ref):
    t = pl.program_id(0)
    w = w_ref[pl.ds(t, 1), :]
    y_ref[...] = jnp.maximum(x_ref[...] + w, 0.0)


def _ybuild(x, w_type):
    yb = 1000
    nb = N // yb
    return pl.pallas_call(
        _ybuild_body,
        grid=(NT, nb),
        in_specs=[
            pl.BlockSpec((yb, D), lambda t, i: (i, 0)),
            pl.BlockSpec((8, D), lambda t, i: (0, 0)),
        ],
        out_specs=pl.BlockSpec((yb, D), lambda t, i: (t * nb + i, 0)),
        out_shape=jax.ShapeDtypeStruct((NT * N, D), jnp.float32),
    )(x, w_type)


_SC_MESH = plsc.VectorSubcoreMesh(core_axis_name="c", subcore_axis_name="s")


@functools.partial(
    pl.kernel,
    mesh=_SC_MESH,
    out_type=jax.ShapeDtypeStruct((N, D), jnp.float32),
    scratch_types=[
        pltpu.VMEM((BPT * K + K,), jnp.int32),
        pltpu.VMEM((2, K), jnp.int32),
        pltpu.VMEM((2, K), jnp.int32),
        pltpu.VMEM((2, K, D), jnp.float32),
        pltpu.VMEM_SHARED((ACC_ROWS, D), jnp.float32),
        pltpu.SemaphoreType.DMA,
        pltpu.SemaphoreType.DMA,
        pltpu.SemaphoreType.DMA,
        pltpu.SemaphoreType.DMA,
    ],
    compiler_params=pltpu.CompilerParams(use_tc_tiling_on_sc=False,
                                        needs_layout_passes=False,
                                        internal_scratch_in_bytes=128 * 1024),
)
def _sc_agg(y_hbm, pix_hbm, out_hbm,
            pix_v, gixb, dixb, rows_v, acc_sh, sem0, sem1, sem2, sem3):
    cid = lax.axis_index("c")
    sid = lax.axis_index("s")
    base = cid * HALF

    def zbody(j, c):
        rows_v[0, j // 16, pl.ds((j % 16) * 16, 16)] = jnp.zeros((16,),
                                                                 jnp.float32)
        return c

    lax.fori_loop(0, K * 16, zbody, 0)
    zb = sid * ZCHUNK
    for o in range(0, ZCHUNK - K + 1, K):
        pltpu.sync_copy(rows_v.at[0], acc_sh.at[pl.ds(zb + o, K)])
    rem = ZCHUNK % K
    if rem:
        pltpu.sync_copy(rows_v.at[0, pl.ds(0, rem)],
                        acc_sh.at[pl.ds(zb + ZCHUNK - rem, rem)])
    pltpu.sync_copy(pix_hbm.at[pl.ds(sid * BPT * K, BPT * K)],
                    pix_v.at[pl.ds(0, BPT * K)])

    plsc.subcore_barrier()

    def unpack(b, slot):
        def ubody(w, c):
            v = pix_v[pl.ds(b * K + w * 16, 16)]
            g = v & 0xFFFF
            d = lax.shift_right_logical(v, 16)
            inr = (d >= base) & (d < base + HALF)
            gixb[slot, pl.ds(w * 16, 16)] = g
            dixb[slot, pl.ds(w * 16, 16)] = jnp.where(inr, d - base, TRASH)
            return c

        lax.fori_loop(0, K // 16, ubody, 0)

    gsems = (sem0, sem1)
    ssems = (sem2, sem3)
    gd = [None] * BPT
    sd = [None] * BPT
    for b in range(BPT):
        slot = b % 2
        if b >= 2:
            sd[b - 2].wait()
        unpack(b, slot)
        gd[b] = pltpu.async_copy(y_hbm.at[gixb.at[slot]],
                                 rows_v.at[slot], gsems[slot])
        if b >= 1:
            gd[b - 1].wait()
            sd[b - 1] = pltpu.async_copy(rows_v.at[(b - 1) % 2],
                                         acc_sh.at[dixb.at[(b - 1) % 2]],
                                         ssems[(b - 1) % 2], add=True)
    gd[BPT - 1].wait()
    sd[BPT - 1] = pltpu.async_copy(rows_v.at[(BPT - 1) % 2],
                                   acc_sh.at[dixb.at[(BPT - 1) % 2]],
                                   ssems[(BPT - 1) % 2], add=True)
    sd[BPT - 2].wait()
    sd[BPT - 1].wait()
    plsc.subcore_barrier()
    start = jnp.minimum(sid * WCHUNK, HALF - WCHUNK)
    pltpu.sync_copy(acc_sh.at[pl.ds(start, WCHUNK)],
                    out_hbm.at[pl.ds(cid * HALF + start, WCHUNK)])


def _mlp_body(a_ref, w1_ref, b1_ref, w2_ref, b2_ref, h_ref, st_ref):
    i = pl.program_id(0)
    a = a_ref[...]
    h1 = lax.dot_general(a, w1_ref[...], (((1,), (1,)), ((), ())),
                         preferred_element_type=jnp.float32)
    h1 = jnp.maximum(h1 + b1_ref[...], 0.0)
    h = lax.dot_general(h1, w2_ref[...], (((1,), (1,)), ((), ())),
                        preferred_element_type=jnp.float32)
    h = h + b2_ref[...]
    h_ref[...] = h

    @pl.when(i == 0)
    def _():
        st_ref[...] = jnp.zeros_like(st_ref)

    st_ref[0:1, :] += jnp.sum(h, axis=0, keepdims=True)
    st_ref[1:2, :] += jnp.sum(h * h, axis=0, keepdims=True)


def _mlp(aggr, w1, b1, w2, b2):
    return pl.pallas_call(
        _mlp_body,
        grid=(N // MB,),
        in_specs=[
            pl.BlockSpec((MB, D), lambda i: (i, 0)),
            pl.BlockSpec((2 * D, D), lambda i: (0, 0)),
            pl.BlockSpec((1, 2 * D), lambda i: (0, 0)),
            pl.BlockSpec((D, 2 * D), lambda i: (0, 0)),
            pl.BlockSpec((1, D), lambda i: (0, 0)),
        ],
        out_specs=[
            pl.BlockSpec((MB, D), lambda i: (i, 0)),
            pl.BlockSpec((8, D), lambda i: (0, 0)),
        ],
        out_shape=[
            jax.ShapeDtypeStruct((N, D), jnp.float32),
            jax.ShapeDtypeStruct((8, D), jnp.float32),
        ],
    )(aggr, w1, b1, w2, b2)


def _norm_body(h_ref, st_ref, gam_ref, bet_ref, o_ref):
    mean = st_ref[0:1, :] * (1.0 / N)
    var = st_ref[1:2, :] * (1.0 / N) - mean * mean
    inv = lax.rsqrt(var + EPS)
    o_ref[...] = jnp.maximum(
        (h_ref[...] - mean) * inv * gam_ref[...] + bet_ref[...], 0.0)


def _norm(h, st, gamma, beta):
    return pl.pallas_call(
        _norm_body,
        grid=(N // MB,),
        in_specs=[
            pl.BlockSpec((MB, D), lambda i: (i, 0)),
            pl.BlockSpec((8, D), lambda i: (0, 0)),
            pl.BlockSpec((1, D), lambda i: (0, 0)),
            pl.BlockSpec((1, D), lambda i: (0, 0)),
        ],
        out_specs=pl.BlockSpec((MB, D), lambda i: (i, 0)),
        out_shape=jax.ShapeDtypeStruct((N, D), jnp.float32),
    )(h, st, gamma, beta)


def _prep(x, edge_index, edge_attr, w_type):
    ei = edge_index.astype(jnp.int32)
    t = edge_attr[:, 0].astype(jnp.int32)
    loops = jnp.arange(N, dtype=jnp.int32)
    npad = EPAD - E - N
    padz = jnp.zeros((npad,), jnp.int32)
    srcp = jnp.concatenate([ei[0], loops, padz]).reshape(EROWS, K)
    dstp = jnp.concatenate(
        [ei[1], loops, jnp.full((npad,), PADDST, jnp.int32)]).reshape(EROWS, K)
    typp = jnp.concatenate(
        [t, jnp.full((N,), 4, jnp.int32), padz]).reshape(EROWS, K)
    packed = _idx_prep(srcp, dstp, typp).reshape(EPAD)
    y = _ybuild(x, w_type)
    return packed, y


def _post(aggr, w1, b1, w2, b2, gamma, beta):
    h, st = _mlp(aggr, w1, b1.reshape(1, -1), w2, b2.reshape(1, -1))
    return _norm(h, st, gamma.reshape(1, -1), beta.reshape(1, -1))


def kernel(xA, edge_indexA, edge_attrA, xB, edge_indexB, edge_attrB,
           W_type, W1, b1, W2, b2, gamma, beta):
    packedA, yA = _prep(xA, edge_indexA, edge_attrA, W_type)
    packedB, yB = _prep(xB, edge_indexB, edge_attrB, W_type)
    aggrA = _sc_agg(yA, packedA)
    aggrB = _sc_agg(yB, packedB)
    outA = _post(aggrA, W1, b1, W2, b2, gamma, beta)
    outB = _post(aggrB, W1, b1, W2, b2, gamma, beta)
    return (outA, outB)
